# baseline (speedup 1.0000x reference)
import dataclasses

import jax
import jax.numpy as jnp
from jax import lax
from jax.experimental import pallas as pl
from jax.experimental.pallas import tpu as pltpu
from jax.experimental.pallas import tpu_sc as plsc

B = 16384
F = 64
L = 16
NC, NS = 2, 16
NW = NC * NS
BPW = B // NW
SR = 8
CH = 16
NCHUNK = BPW // CH


def _enqueue_groups(idx_ref, tab_hbm, dst, sem, b0, buf):
    for g in range(0, CH, L):
        vec = idx_ref[pl.ds(b0 + g, L)]
        for j in range(L):
            pltpu.async_copy(tab_hbm.at[vec[j] >> 3], dst.at[buf, g + j], sem)


def _gather_user_body(users_hbm, ue_hbm, gu_hbm,
                      uidx_v, ue_v, st_v, sem_u, sem_st):
    wid = lax.axis_index("s") * NC + lax.axis_index("c")
    base = wid * BPW
    nc = F // L

    pltpu.sync_copy(users_hbm.at[pl.ds(base, BPW)], uidx_v)
    _enqueue_groups(uidx_v, ue_hbm, ue_v, sem_u, 0, 0)

    @pl.loop(0, NCHUNK)
    def _(t):
        b0 = t * CH
        cb = t & 1

        @pl.when(t + 1 < NCHUNK)
        def _():
            _enqueue_groups(uidx_v, ue_hbm, ue_v, sem_u, b0 + CH,
                            (t + 1) & 1)

        @pl.when(t >= 2)
        def _():
            pltpu.make_async_copy(st_v.at[0], gu_hbm.at[pl.ds(0, CH)],
                                  sem_st).wait()

        pltpu.make_async_copy(ue_hbm.at[pl.ds(0, CH)], ue_v.at[0],
                              sem_u).wait()

        for g in range(0, CH, L):
            uvec = uidx_v[pl.ds(b0 + g, L)]
            ur = uvec & (SR - 1)
            for j in range(L):
                su = ur[j]
                for c in range(nc):
                    st_v[cb, g + j, pl.ds(c * L, L)] = (
                        ue_v[cb, g + j, su, pl.ds(c * L, L)])

        pltpu.async_copy(st_v.at[cb], gu_hbm.at[pl.ds(base + b0, CH)],
                         sem_st)

    @pl.loop(0, 2)
    def _(t):
        pltpu.make_async_copy(st_v.at[0], gu_hbm.at[pl.ds(0, CH)],
                              sem_st).wait()


def _dot_item_body(users_hbm, items_hbm, ie_hbm, gu_hbm, ub_hbm, ib_hbm,
                   out_hbm, uidx_v, iidx_v, ie_v, gu_v, ub_v, ib_v, out_v,
                   sem_i, sem_g, sem_ub, sem_ib):
    wid = lax.axis_index("s") * NC + lax.axis_index("c")
    base = wid * BPW
    nc = F // L
    lane = lax.broadcasted_iota(jnp.int32, (L,), 0)

    pltpu.sync_copy(users_hbm.at[pl.ds(base, BPW)], uidx_v)
    pltpu.sync_copy(items_hbm.at[pl.ds(base, BPW)], iidx_v)

    cub = pltpu.async_copy(ub_hbm.at[uidx_v], ub_v, sem_ub)
    cib = pltpu.async_copy(ib_hbm.at[iidx_v], ib_v, sem_ib)

    _enqueue_groups(iidx_v, ie_hbm, ie_v, sem_i, 0, 0)
    pltpu.async_copy(gu_hbm.at[pl.ds(base, CH)], gu_v.at[0], sem_g)

    @pl.loop(0, NCHUNK)
    def _(t):
        b0 = t * CH
        cb = t & 1

        @pl.when(t + 1 < NCHUNK)
        def _():
            _enqueue_groups(iidx_v, ie_hbm, ie_v, sem_i, b0 + CH,
                            (t + 1) & 1)
            pltpu.async_copy(gu_hbm.at[pl.ds(base + b0 + CH, CH)],
                             gu_v.at[(t + 1) & 1], sem_g)

        pltpu.make_async_copy(ie_hbm.at[pl.ds(0, CH)], ie_v.at[0],
                              sem_i).wait()
        pltpu.make_async_copy(gu_hbm.at[pl.ds(0, CH)], gu_v.at[0],
                              sem_g).wait()

        for g in range(0, CH, L):
            ivec = iidx_v[pl.ds(b0 + g, L)]
            ir = ivec & (SR - 1)

            res = jnp.zeros((L,), jnp.float32)
            for j in range(L):
                si = ir[j]
                acc = (gu_v[cb, g + j, pl.ds(0, L)]
                       * ie_v[cb, g + j, si, pl.ds(0, L)])
                for c in range(1, nc):
                    acc = acc + (gu_v[cb, g + j, pl.ds(c * L, L)]
                                 * ie_v[cb, g + j, si, pl.ds(c * L, L)])
                res = jnp.where(lane == j, jnp.sum(acc), res)
            out_v[pl.ds(b0 + g, L)] = res

    cub.wait()
    cib.wait()

    @pl.loop(0, BPW, step=L)
    def _(g):
        out_v[pl.ds(g, L)] = (out_v[pl.ds(g, L)] + ub_v[pl.ds(g, L)]
                              + ib_v[pl.ds(g, L)])

    pltpu.sync_copy(out_v, out_hbm.at[pl.ds(base, BPW)])


def kernel(users, items, user_emb, user_bias, item_emb, item_bias):
    mesh = plsc.VectorSubcoreMesh(core_axis_name="c", subcore_axis_name="s")
    cp = pltpu.CompilerParams()
    if "needs_layout_passes" in pltpu.CompilerParams.__dataclass_fields__:
        cp = dataclasses.replace(cp, needs_layout_passes=False)

    k_user = pl.kernel(
        _gather_user_body,
        out_type=jax.ShapeDtypeStruct((B, F), jnp.float32),
        mesh=mesh,
        compiler_params=cp,
        scratch_types=[
            pltpu.VMEM((BPW,), jnp.int32),
            pltpu.VMEM((2, CH, SR, F), jnp.float32),
            pltpu.VMEM((2, CH, F), jnp.float32),
            pltpu.SemaphoreType.DMA,
            pltpu.SemaphoreType.DMA,
        ],
    )
    k_item = pl.kernel(
        _dot_item_body,
        out_type=jax.ShapeDtypeStruct((B,), jnp.float32),
        mesh=mesh,
        compiler_params=cp,
        scratch_types=[
            pltpu.VMEM((BPW,), jnp.int32),
            pltpu.VMEM((BPW,), jnp.int32),
            pltpu.VMEM((2, CH, SR, F), jnp.float32),
            pltpu.VMEM((2, CH, F), jnp.float32),
            pltpu.VMEM((BPW,), jnp.float32),
            pltpu.VMEM((BPW,), jnp.float32),
            pltpu.VMEM((BPW,), jnp.float32),
            pltpu.SemaphoreType.DMA,
            pltpu.SemaphoreType.DMA,
            pltpu.SemaphoreType.DMA,
            pltpu.SemaphoreType.DMA,
        ],
    )

    users = users.astype(jnp.int32)
    items = items.astype(jnp.int32)
    n_users = user_emb.shape[0]
    n_items = item_emb.shape[0]
    ue3 = user_emb.reshape(n_users // SR, SR, F)
    ie3 = item_emb.reshape(n_items // SR, SR, F)
    gu = k_user(users, ue3)
    return k_item(users, items, ie3, gu,
                  user_bias.reshape(-1), item_bias.reshape(-1))


# speedup vs baseline: 1.0351x; 1.0351x over previous
import dataclasses

import jax
import jax.numpy as jnp
from jax import lax
from jax.experimental import pallas as pl
from jax.experimental.pallas import tpu as pltpu
from jax.experimental.pallas import tpu_sc as plsc

B = 16384
F = 64
L = 16
NC, NS = 2, 16
NW = NC * NS
BPW = B // NW
SR = 8
CH = 16
NCHUNK = BPW // CH


def _cf_body(users_hbm, items_hbm, ue_hbm, ub_hbm, ie_hbm, ib_hbm, out_hbm,
             uidx_v, iidx_v, ue_v, ie_v, ub_v, ib_v, out_v,
             sem_u, sem_i, sem_ub, sem_ib):
    wid = lax.axis_index("s") * NC + lax.axis_index("c")
    base = wid * BPW

    pltpu.sync_copy(users_hbm.at[pl.ds(base, BPW)], uidx_v)
    pltpu.sync_copy(items_hbm.at[pl.ds(base, BPW)], iidx_v)

    cub = pltpu.async_copy(ub_hbm.at[uidx_v], ub_v, sem_ub)
    cib = pltpu.async_copy(ib_hbm.at[iidx_v], ib_v, sem_ib)

    lane = lax.broadcasted_iota(jnp.int32, (L,), 0)
    nc = F // L

    def enqueue_chunk(tc, buf):
        b0 = tc * CH
        for g in range(0, CH, L):
            uvec = uidx_v[pl.ds(b0 + g, L)]
            ivec = iidx_v[pl.ds(b0 + g, L)]
            for j in range(L):
                ru = uvec[j] >> 3
                ri = ivec[j] >> 3
                pltpu.async_copy(ue_hbm.at[ru], ue_v.at[buf, g + j], sem_u)
                pltpu.async_copy(ie_hbm.at[ri], ie_v.at[buf, g + j], sem_i)

    enqueue_chunk(0, 0)
    enqueue_chunk(1, 1)

    @pl.loop(0, NCHUNK)
    def _(t):
        b0 = t * CH
        cb = t % 3

        @pl.when(t + 2 < NCHUNK)
        def _():
            enqueue_chunk(t + 2, (t + 2) % 3)

        pltpu.make_async_copy(ue_hbm.at[pl.ds(0, CH)], ue_v.at[0],
                              sem_u).wait()
        pltpu.make_async_copy(ie_hbm.at[pl.ds(0, CH)], ie_v.at[0],
                              sem_i).wait()

        for g in range(0, CH, L):
            uvec = uidx_v[pl.ds(b0 + g, L)]
            ivec = iidx_v[pl.ds(b0 + g, L)]
            ur = uvec & (SR - 1)
            ir = ivec & (SR - 1)

            res = jnp.zeros((L,), jnp.float32)
            for j in range(L):
                su = ur[j]
                si = ir[j]
                acc = (ue_v[cb, g + j, su, pl.ds(0, L)]
                       * ie_v[cb, g + j, si, pl.ds(0, L)])
                for c in range(1, nc):
                    acc = acc + (ue_v[cb, g + j, su, pl.ds(c * L, L)]
                                 * ie_v[cb, g + j, si, pl.ds(c * L, L)])
                res = jnp.where(lane == j, jnp.sum(acc), res)
            out_v[pl.ds(b0 + g, L)] = res

    cub.wait()
    cib.wait()

    @pl.loop(0, BPW, step=L)
    def _(g):
        out_v[pl.ds(g, L)] = (out_v[pl.ds(g, L)] + ub_v[pl.ds(g, L)]
                              + ib_v[pl.ds(g, L)])

    pltpu.sync_copy(out_v, out_hbm.at[pl.ds(base, BPW)])


def kernel(users, items, user_emb, user_bias, item_emb, item_bias):
    mesh = plsc.VectorSubcoreMesh(core_axis_name="c", subcore_axis_name="s")
    cp = pltpu.CompilerParams()
    if "needs_layout_passes" in pltpu.CompilerParams.__dataclass_fields__:
        cp = dataclasses.replace(cp, needs_layout_passes=False)
    k = pl.kernel(
        _cf_body,
        out_type=jax.ShapeDtypeStruct((B,), jnp.float32),
        mesh=mesh,
        compiler_params=cp,
        scratch_types=[
            pltpu.VMEM((BPW,), jnp.int32),
            pltpu.VMEM((BPW,), jnp.int32),
            pltpu.VMEM((3, CH, SR, F), jnp.float32),
            pltpu.VMEM((3, CH, SR, F), jnp.float32),
            pltpu.VMEM((BPW,), jnp.float32),
            pltpu.VMEM((BPW,), jnp.float32),
            pltpu.VMEM((BPW,), jnp.float32),
            pltpu.SemaphoreType.DMA,
            pltpu.SemaphoreType.DMA,
            pltpu.SemaphoreType.DMA,
            pltpu.SemaphoreType.DMA,
        ],
    )
    n_users = user_emb.shape[0]
    n_items = item_emb.shape[0]
    return k(users.astype(jnp.int32), items.astype(jnp.int32),
             user_emb.reshape(n_users // SR, SR, F), user_bias.reshape(-1),
             item_emb.reshape(n_items // SR, SR, F), item_bias.reshape(-1))
